# baseline (speedup 1.0000x reference)
import jax
import jax.numpy as jnp
from jax.experimental import pallas as pl
from jax.experimental.pallas import tpu as pltpu


_TILE_I = 512


def _hausdorff_kernel(p1_ref, p2_ref, out_ref, dist2_ref, m1_ref):
    b = pl.program_id(0)
    i = pl.program_id(1)
    ni = pl.num_programs(1)

    @pl.when(jnp.logical_and(b == 0, i == 0))
    def _init_out():
        out_ref[0:1, 0:1] = jnp.zeros((1, 1), jnp.float32)

    @pl.when(i == 0)
    def _init_batch():
        m1_ref[0, 0] = -jnp.inf
        dist2_ref[0:1, :] = jnp.full((1, dist2_ref.shape[1]), jnp.inf,
                                     dtype=jnp.float32)

    a = p1_ref[0]
    c = p2_ref[0]

    asq = a * a
    n1 = asq[0:1] + asq[1:2] + asq[2:3]
    csq = c * c
    n2 = csq[0:1] + csq[1:2] + csq[2:3]

    lhs = jnp.concatenate([a, n1, jnp.ones_like(n1)], axis=0)
    rhs = jnp.concatenate([-2.0 * c, jnp.ones_like(n2), n2], axis=0)

    d = jax.lax.dot_general(
        lhs, rhs, (((0,), (0,)), ((), ())),
        preferred_element_type=jnp.float32,
        precision=jax.lax.Precision.HIGHEST)

    row_min = jnp.min(d, axis=1)
    m1_ref[0, 0] = jnp.maximum(m1_ref[0, 0], jnp.max(row_min))

    col_min = jnp.min(d, axis=0, keepdims=True)
    dist2_ref[0:1, :] = jnp.minimum(dist2_ref[0:1, :], col_min)

    @pl.when(i == ni - 1)
    def _finish_batch():
        m2 = jnp.max(dist2_ref[0:1, :])
        out_ref[0:1, 0:1] = out_ref[0:1, 0:1] + jnp.maximum(m1_ref[0, 0], m2)


def kernel(points1, points2):
    bsz, n1, _ = points1.shape
    _, n2, _ = points2.shape
    p1t = jnp.transpose(points1, (0, 2, 1))
    p2t = jnp.transpose(points2, (0, 2, 1))

    ni = n1 // _TILE_I
    out = pl.pallas_call(
        _hausdorff_kernel,
        grid=(bsz, ni),
        in_specs=[
            pl.BlockSpec((1, 3, _TILE_I), lambda b, i: (b, 0, i)),
            pl.BlockSpec((1, 3, n2), lambda b, i: (b, 0, 0)),
        ],
        out_specs=pl.BlockSpec((1, 1), lambda b, i: (0, 0)),
        out_shape=jax.ShapeDtypeStruct((1, 1), jnp.float32),
        scratch_shapes=[
            pltpu.VMEM((1, n2), jnp.float32),
            pltpu.SMEM((1, 1), jnp.float32),
        ],
    )(p1t, p2t)
    return out[0, 0]


# speedup vs baseline: 4.2578x; 4.2578x over previous
import jax
import jax.numpy as jnp
from jax.experimental import pallas as pl
from jax.experimental.pallas import tpu as pltpu


_TILE_I = 512


def _hausdorff_kernel(p1_ref, p2_ref, out_ref, dist2_ref, m1_ref):
    b = pl.program_id(0)
    i = pl.program_id(1)
    ni = pl.num_programs(1)

    @pl.when(jnp.logical_and(b == 0, i == 0))
    def _init_out():
        out_ref[0:1, 0:1] = jnp.zeros((1, 1), jnp.float32)

    @pl.when(i == 0)
    def _init_batch():
        m1_ref[0, 0] = -jnp.inf
        dist2_ref[0:1, :] = jnp.full((1, dist2_ref.shape[1]), jnp.inf,
                                     dtype=jnp.float32)

    a = p1_ref[0]
    c = p2_ref[0]

    asq = a * a
    n1 = asq[0:1] + asq[1:2] + asq[2:3]
    csq = c * c
    n2 = csq[0:1] + csq[1:2] + csq[2:3]

    lhs = jnp.concatenate([a, n1, jnp.ones_like(n1)], axis=0)
    rhs = jnp.concatenate([-2.0 * c, jnp.ones_like(n2), n2], axis=0)

    lhs_hi = lhs.astype(jnp.bfloat16)
    lhs_lo = (lhs - lhs_hi.astype(jnp.float32)).astype(jnp.bfloat16)
    rhs_hi = rhs.astype(jnp.bfloat16)
    rhs_lo = (rhs - rhs_hi.astype(jnp.float32)).astype(jnp.bfloat16)
    lhs15 = jnp.concatenate([lhs_hi, lhs_hi, lhs_lo], axis=0)
    rhs15 = jnp.concatenate([rhs_hi, rhs_lo, rhs_hi], axis=0)

    d = jax.lax.dot_general(
        lhs15, rhs15, (((0,), (0,)), ((), ())),
        preferred_element_type=jnp.float32)

    row_min = jnp.min(d, axis=1)
    m1_ref[0, 0] = jnp.maximum(m1_ref[0, 0], jnp.max(row_min))

    col_min = jnp.min(d, axis=0, keepdims=True)
    dist2_ref[0:1, :] = jnp.minimum(dist2_ref[0:1, :], col_min)

    @pl.when(i == ni - 1)
    def _finish_batch():
        m2 = jnp.max(dist2_ref[0:1, :])
        out_ref[0:1, 0:1] = out_ref[0:1, 0:1] + jnp.maximum(m1_ref[0, 0], m2)


def kernel(points1, points2):
    bsz, n1, _ = points1.shape
    _, n2, _ = points2.shape
    p1t = jnp.transpose(points1, (0, 2, 1))
    p2t = jnp.transpose(points2, (0, 2, 1))

    ni = n1 // _TILE_I
    out = pl.pallas_call(
        _hausdorff_kernel,
        grid=(bsz, ni),
        in_specs=[
            pl.BlockSpec((1, 3, _TILE_I), lambda b, i: (b, 0, i)),
            pl.BlockSpec((1, 3, n2), lambda b, i: (b, 0, 0)),
        ],
        out_specs=pl.BlockSpec((1, 1), lambda b, i: (0, 0)),
        out_shape=jax.ShapeDtypeStruct((1, 1), jnp.float32),
        scratch_shapes=[
            pltpu.VMEM((1, n2), jnp.float32),
            pltpu.SMEM((1, 1), jnp.float32),
        ],
    )(p1t, p2t)
    return out[0, 0]


# speedup vs baseline: 5.0097x; 1.1766x over previous
import jax
import jax.numpy as jnp
from jax.experimental import pallas as pl
from jax.experimental.pallas import tpu as pltpu


_TILE_I = 2048


def _hausdorff_kernel(a_ref, b_ref, out_ref, dist2_ref, m1_ref):
    b = pl.program_id(0)
    i = pl.program_id(1)
    ni = pl.num_programs(1)

    @pl.when(jnp.logical_and(b == 0, i == 0))
    def _init_out():
        out_ref[0:1, 0:1] = jnp.zeros((1, 1), jnp.float32)

    @pl.when(i == 0)
    def _init_batch():
        m1_ref[0, 0] = -jnp.inf
        dist2_ref[0:1, :] = jnp.full((1, dist2_ref.shape[1]), jnp.inf,
                                     dtype=jnp.float32)

    d = jax.lax.dot_general(
        a_ref[0], b_ref[0], (((0,), (0,)), ((), ())),
        preferred_element_type=jnp.float32)

    row_min = jnp.min(d, axis=1)
    m1_ref[0, 0] = jnp.maximum(m1_ref[0, 0], jnp.max(row_min))

    col_min = jnp.min(d, axis=0, keepdims=True)
    dist2_ref[0:1, :] = jnp.minimum(dist2_ref[0:1, :], col_min)

    @pl.when(i == ni - 1)
    def _finish_batch():
        m2 = jnp.max(dist2_ref[0:1, :])
        out_ref[0:1, 0:1] = out_ref[0:1, 0:1] + jnp.maximum(m1_ref[0, 0], m2)


def _augment(pts, negate):
    pt = jnp.transpose(pts, (0, 2, 1))
    n = jnp.sum(pt * pt, axis=1, keepdims=True)
    ones = jnp.ones_like(n)
    if negate:
        full = jnp.concatenate([-2.0 * pt, ones, n], axis=1)
    else:
        full = jnp.concatenate([pt, n, ones], axis=1)
    hi = full.astype(jnp.bfloat16)
    lo = (full - hi.astype(jnp.float32)).astype(jnp.bfloat16)
    if negate:
        return jnp.concatenate([hi, lo, hi], axis=1)
    return jnp.concatenate([hi, hi, lo], axis=1)


def kernel(points1, points2):
    bsz, n1, _ = points1.shape
    _, n2, _ = points2.shape
    a15 = _augment(points1, negate=False)
    b15 = _augment(points2, negate=True)

    ni = n1 // _TILE_I
    out = pl.pallas_call(
        _hausdorff_kernel,
        grid=(bsz, ni),
        in_specs=[
            pl.BlockSpec((1, 15, _TILE_I), lambda b, i: (b, 0, i)),
            pl.BlockSpec((1, 15, n2), lambda b, i: (b, 0, 0)),
        ],
        out_specs=pl.BlockSpec((1, 1), lambda b, i: (0, 0)),
        out_shape=jax.ShapeDtypeStruct((1, 1), jnp.float32),
        scratch_shapes=[
            pltpu.VMEM((1, n2), jnp.float32),
            pltpu.SMEM((1, 1), jnp.float32),
        ],
    )(a15, b15)
    return out[0, 0]


# speedup vs baseline: 5.0819x; 1.0144x over previous
import jax
import jax.numpy as jnp
from jax.experimental import pallas as pl
from jax.experimental.pallas import tpu as pltpu


_TILE_I = 2048


def _split15(x, flip):
    hi = x.astype(jnp.bfloat16)
    lo = (x - hi.astype(jnp.float32)).astype(jnp.bfloat16)
    if flip:
        return jnp.concatenate([hi, lo, hi], axis=0)
    return jnp.concatenate([hi, hi, lo], axis=0)


def _hausdorff_kernel(a_ref, b_ref, out_ref, dist2_ref, m1_ref):
    b = pl.program_id(0)
    i = pl.program_id(1)
    ni = pl.num_programs(1)

    @pl.when(jnp.logical_and(b == 0, i == 0))
    def _init_out():
        out_ref[0:1, 0:1] = jnp.zeros((1, 1), jnp.float32)

    @pl.when(i == 0)
    def _init_batch():
        m1_ref[0, 0] = -jnp.inf
        dist2_ref[0:1, :] = jnp.full((1, dist2_ref.shape[1]), jnp.inf,
                                     dtype=jnp.float32)

    lhs15 = _split15(a_ref[0], flip=False)
    rhs15 = _split15(b_ref[0], flip=True)

    d = jax.lax.dot_general(
        lhs15, rhs15, (((0,), (0,)), ((), ())),
        preferred_element_type=jnp.float32)

    row_min = jnp.min(d, axis=1)
    m1_ref[0, 0] = jnp.maximum(m1_ref[0, 0], jnp.max(row_min))

    col_min = jnp.min(d, axis=0, keepdims=True)
    dist2_ref[0:1, :] = jnp.minimum(dist2_ref[0:1, :], col_min)

    @pl.when(i == ni - 1)
    def _finish_batch():
        m2 = jnp.max(dist2_ref[0:1, :])
        out_ref[0:1, 0:1] = out_ref[0:1, 0:1] + jnp.maximum(m1_ref[0, 0], m2)


def _augment(pts, rhs_layout):
    pt = jnp.transpose(pts, (0, 2, 1))
    n = jnp.sum(pt * pt, axis=1, keepdims=True)
    ones = jnp.ones_like(n)
    if rhs_layout:
        return jnp.concatenate([-2.0 * pt, ones, n], axis=1)
    return jnp.concatenate([pt, n, ones], axis=1)


def kernel(points1, points2):
    bsz, n1, _ = points1.shape
    _, n2, _ = points2.shape
    a5 = _augment(points1, rhs_layout=False)
    b5 = _augment(points2, rhs_layout=True)

    ni = n1 // _TILE_I
    out = pl.pallas_call(
        _hausdorff_kernel,
        grid=(bsz, ni),
        in_specs=[
            pl.BlockSpec((1, 5, _TILE_I), lambda b, i: (b, 0, i)),
            pl.BlockSpec((1, 5, n2), lambda b, i: (b, 0, 0)),
        ],
        out_specs=pl.BlockSpec((1, 1), lambda b, i: (0, 0)),
        out_shape=jax.ShapeDtypeStruct((1, 1), jnp.float32),
        scratch_shapes=[
            pltpu.VMEM((1, n2), jnp.float32),
            pltpu.SMEM((1, 1), jnp.float32),
        ],
    )(a5, b5)
    return out[0, 0]


# speedup vs baseline: 5.1538x; 1.0141x over previous
import jax
import jax.numpy as jnp
from jax.experimental import pallas as pl
from jax.experimental.pallas import tpu as pltpu


_TILE_I = 2048


def _split15(x, flip):
    hi = x.astype(jnp.bfloat16)
    lo = (x - hi.astype(jnp.float32)).astype(jnp.bfloat16)
    if flip:
        return jnp.concatenate([hi, lo, hi], axis=0)
    return jnp.concatenate([hi, hi, lo], axis=0)


def _hausdorff_kernel(a_ref, b_ref, out_ref, dist2_ref, m1_ref):
    b = pl.program_id(0)
    i = pl.program_id(1)
    ni = pl.num_programs(1)

    @pl.when(jnp.logical_and(b == 0, i == 0))
    def _init_out():
        out_ref[0:1, 0:1] = jnp.zeros((1, 1), jnp.float32)

    @pl.when(i == 0)
    def _init_batch():
        m1_ref[0, 0] = -jnp.inf
        dist2_ref[0:1, :] = jnp.full((1, dist2_ref.shape[1]), jnp.inf,
                                     dtype=jnp.float32)

    lhs15 = _split15(a_ref[0], flip=False)
    rhs15 = _split15(b_ref[0], flip=True)

    d = jax.lax.dot_general(
        lhs15, rhs15, (((0,), (0,)), ((), ())),
        preferred_element_type=jnp.float32)

    row_min = jnp.min(d, axis=1)
    m1_ref[0, 0] = jnp.maximum(m1_ref[0, 0], jnp.max(row_min))

    col_min = jnp.min(d, axis=0, keepdims=True)
    dist2_ref[0:1, :] = jnp.minimum(dist2_ref[0:1, :], col_min)

    @pl.when(i == ni - 1)
    def _finish_batch():
        m2 = jnp.max(dist2_ref[0:1, :])
        out_ref[0:1, 0:1] = out_ref[0:1, 0:1] + jnp.maximum(m1_ref[0, 0], m2)


def kernel(points1, points2):
    bsz, n1, _ = points1.shape
    _, n2, _ = points2.shape

    pts = jnp.concatenate([points1, points2], axis=0)
    nn = jnp.sum(pts * pts, axis=2, keepdims=True)
    ones = jnp.ones_like(nn)
    is_lhs = (jnp.arange(2 * bsz) < bsz).reshape(-1, 1, 1)
    aug = jnp.concatenate(
        [jnp.where(is_lhs, pts, -2.0 * pts),
         jnp.where(is_lhs, nn, ones),
         jnp.where(is_lhs, ones, nn)], axis=2)
    ab5 = jnp.transpose(aug, (0, 2, 1))

    ni = n1 // _TILE_I
    out = pl.pallas_call(
        _hausdorff_kernel,
        grid=(bsz, ni),
        in_specs=[
            pl.BlockSpec((1, 5, _TILE_I), lambda b, i: (b, 0, i)),
            pl.BlockSpec((1, 5, n2), lambda b, i, bsz=bsz: (b + bsz, 0, 0)),
        ],
        out_specs=pl.BlockSpec((1, 1), lambda b, i: (0, 0)),
        out_shape=jax.ShapeDtypeStruct((1, 1), jnp.float32),
        scratch_shapes=[
            pltpu.VMEM((1, n2), jnp.float32),
            pltpu.SMEM((1, 1), jnp.float32),
        ],
    )(ab5, ab5)
    return out[0, 0]
